# baseline (speedup 1.0000x reference)
import jax
import jax.numpy as jnp
from jax.experimental import pallas as pl
from jax.experimental.pallas import tpu as pltpu

EPS = 0.001
SUPPORT = 300.0
BINS = 601
LANES = 512
BLOCK_H = 256


def _transform(x):
    x = jnp.where(jnp.isnan(x) | jnp.isinf(x), 0.0, x)
    t = jnp.sign(x) * (jnp.sqrt(jnp.abs(x) + 1.0) - 1.0) + EPS * x
    return jnp.clip(t, -SUPPORT, SUPPORT)


def _twohot_flat_block(sc_lo_ref, sc_hi_ref, p_lo_ref, p_hi_ref, out_ref):
    a_lo = _transform(sc_lo_ref[0, 0, :]) + p_lo_ref[0, 0, :]
    a_hi = _transform(sc_hi_ref[0, 0, :]) + p_hi_ref[0, 0, :]
    c = jax.lax.broadcasted_iota(jnp.int32, (BLOCK_H, LANES), 1).astype(
        jnp.float32
    )
    d1 = jnp.abs(a_lo[:, None] - c)
    d2 = jnp.abs(a_hi[:, None] - c)
    out_ref[:, :] = jnp.maximum(1.0 - jnp.minimum(d1, d2), 0.0)


@jax.jit
def _twohot(scalar):
    n = scalar.shape[0]
    rows2d = n * BINS // LANES
    grid = rows2d // BLOCK_H
    flat0 = jnp.arange(rows2d, dtype=jnp.int32) * LANES
    i0 = flat0 // BINS
    j0f = (flat0 - i0 * BINS).astype(jnp.float32)
    sc_lo = scalar[i0].reshape(grid, 1, BLOCK_H)
    sc_hi = scalar[jnp.minimum(i0 + 1, n - 1)].reshape(grid, 1, BLOCK_H)
    p_lo = (SUPPORT - j0f).reshape(grid, 1, BLOCK_H)
    p_hi = (SUPPORT + BINS - j0f).reshape(grid, 1, BLOCK_H)
    out = pl.pallas_call(
        _twohot_flat_block,
        grid=(grid,),
        in_specs=[pl.BlockSpec((1, 1, BLOCK_H), lambda i: (i, 0, 0))] * 4,
        out_specs=pl.BlockSpec((BLOCK_H, LANES), lambda i: (i, 0)),
        out_shape=jax.ShapeDtypeStruct((rows2d, LANES), jnp.float32),
        compiler_params=pltpu.CompilerParams(
            dimension_semantics=("arbitrary",),
        ),
    )(sc_lo, sc_hi, p_lo, p_hi)
    return out.reshape(n, BINS)


def kernel(scalar, support_size):
    return _twohot(scalar)


# speedup vs baseline: 3.0232x; 3.0232x over previous
import jax
import jax.numpy as jnp
from jax.experimental import pallas as pl
from jax.experimental.pallas import tpu as pltpu

EPS = 0.001
SUPPORT = 300.0
BINS = 601
BINS_PAD = 640
ROWS_PER_BLOCK = 512


def _twohot_block(scalar_ref, out_ref):
    x = scalar_ref[0, 0, :]
    x = jnp.where(jnp.isnan(x) | jnp.isinf(x), 0.0, x)
    t = jnp.sign(x) * (jnp.sqrt(jnp.abs(x) + 1.0) - 1.0) + EPS * x
    shifted = jnp.clip(t, -SUPPORT, SUPPORT) + SUPPORT
    colf = jax.lax.broadcasted_iota(
        jnp.int32, (x.shape[0], BINS_PAD), 1
    ).astype(jnp.float32)
    out_ref[:, :] = jnp.maximum(1.0 - jnp.abs(shifted[:, None] - colf), 0.0)


@jax.jit
def _twohot(scalar):
    n = scalar.shape[0]
    nblocks = n // ROWS_PER_BLOCK
    scalar3d = scalar.reshape(nblocks, 1, ROWS_PER_BLOCK)
    padded = pl.pallas_call(
        _twohot_block,
        grid=(nblocks,),
        in_specs=[pl.BlockSpec((1, 1, ROWS_PER_BLOCK), lambda i: (i, 0, 0))],
        out_specs=pl.BlockSpec((ROWS_PER_BLOCK, BINS_PAD), lambda i: (i, 0)),
        out_shape=jax.ShapeDtypeStruct((n, BINS_PAD), jnp.float32),
        compiler_params=pltpu.CompilerParams(
            dimension_semantics=("arbitrary",),
        ),
    )(scalar3d)
    return padded[:, :BINS]


def kernel(scalar, support_size):
    return _twohot(scalar)
